# baseline (speedup 1.0000x reference)
import functools
import math

import jax
import jax.numpy as jnp
from jax import lax
from jax.experimental import pallas as pl
from jax.experimental.pallas import tpu as pltpu
from jax.experimental.pallas import tpu_sc as plsc

D_MODEL = 64
SCALE = math.sqrt(D_MODEL)
NUM_CORES = 2
NUM_SUBCORES = 16
NW = NUM_CORES * NUM_SUBCORES
BT = 128


def _make_embed(V, NB, NR):
    JT = NB // BT
    RT = NR // 8
    n_units = NR
    mesh = plsc.VectorSubcoreMesh(core_axis_name="c", subcore_axis_name="s")

    scratch = (
        [pltpu.VMEM((RT, 8, BT), jnp.int32)]
        + [pltpu.VMEM((BT, D_MODEL), jnp.float32) for _ in range(2)]
        + [pltpu.VMEM((8, 8, BT), jnp.float32) for _ in range(2)]
        + [pltpu.SemaphoreType.DMA for _ in range(4)]
    )

    @functools.partial(
        pl.kernel,
        mesh=mesh,
        compiler_params=pltpu.CompilerParams(
            use_tc_tiling_on_sc=False, needs_layout_passes=False
        ),
        out_type=jax.ShapeDtypeStruct((NR, 8, JT, 8, BT), jnp.float32),
        scratch_types=scratch,
    )
    def k(table_hbm, idx_hbm, out_hbm, idxv, g0, g1, o0, o1, gs0, gs1, os0, os1):
        wid = lax.axis_index("s") * NUM_CORES + lax.axis_index("c")
        gb, ob, gs, osem = (g0, g1), (o0, o1), (gs0, gs1), (os0, os1)
        pltpu.sync_copy(idx_hbm.at[:, wid], idxv)
        iota = lax.iota(jnp.int32, 16)
        rowv = [iota + 16 * t for t in range(8)]

        for p in range(2):
            pltpu.async_copy(table_hbm.at[idxv.at[0, p]], gb[p], gs[p])

        def outer(t2, _):
            for p in range(2):
                u = t2 * 2 + p
                rt, rs = u >> 3, u & 7
                pltpu.make_async_copy(
                    table_hbm.at[idxv.at[rt, rs]], gb[p], gs[p]
                ).wait()

                @pl.when(u >= 2)
                def _():
                    for i in range(8):
                        pltpu.make_async_copy(
                            ob[p].at[i], out_hbm.at[0, i, 0], osem[p]
                        ).wait()

                def body(i, _, p=p):
                    for s in range(8):
                        colv = jnp.full((16,), 8 * i + s, jnp.int32)
                        for t in range(8):
                            v = plsc.load_gather(gb[p], [rowv[t], colv])
                            ob[p][i, s, pl.ds(16 * t, 16)] = v * SCALE
                    return 0

                lax.fori_loop(0, 8, body, 0)
                for i in range(8):
                    pltpu.async_copy(ob[p].at[i], out_hbm.at[u, i, wid], osem[p])

                uu = u + 2

                @pl.when(uu < n_units)
                def _():
                    pltpu.async_copy(
                        table_hbm.at[idxv.at[uu >> 3, uu & 7]], gb[p], gs[p]
                    )
            return 0

        lax.fori_loop(0, n_units // 2, outer, 0)
        for p in range(2):
            for i in range(8):
                pltpu.make_async_copy(
                    ob[p].at[i], out_hbm.at[0, i, 0], osem[p]
                ).wait()

    return k


def kernel(x, table):
    NB, NR = x.shape
    V = table.shape[0]
    JT, RT = NB // BT, NR // 8
    xt = x.T.reshape(RT, 8, JT, BT).transpose(0, 2, 1, 3).astype(jnp.int32)
    L = _make_embed(V, NB, NR)(table, xt)
    return L.transpose(2, 4, 0, 1, 3).reshape(NB, NR, D_MODEL)


# speedup vs baseline: 1.7587x; 1.7587x over previous
import functools
import math

import jax
import jax.numpy as jnp
from jax import lax
from jax.experimental import pallas as pl
from jax.experimental.pallas import tpu as pltpu
from jax.experimental.pallas import tpu_sc as plsc

D_MODEL = 64
SCALE = math.sqrt(D_MODEL)
NUM_CORES = 2
NUM_SUBCORES = 16
NW = NUM_CORES * NUM_SUBCORES
BT = 128


def _make_embed(V, NB, NR):
    JT = NB // BT
    RT = NR // 8
    n_units = NR
    mesh = plsc.VectorSubcoreMesh(core_axis_name="c", subcore_axis_name="s")

    OPAD = BT + 1
    scratch = (
        [pltpu.VMEM((RT, 8, BT), jnp.int32)]
        + [pltpu.VMEM((BT, D_MODEL), jnp.float32) for _ in range(2)]
        + [pltpu.VMEM((D_MODEL, OPAD), jnp.float32) for _ in range(2)]
        + [pltpu.SemaphoreType.DMA for _ in range(4)]
    )

    @functools.partial(
        pl.kernel,
        mesh=mesh,
        compiler_params=pltpu.CompilerParams(
            use_tc_tiling_on_sc=False, needs_layout_passes=False
        ),
        out_type=jax.ShapeDtypeStruct((NR, 8, JT, 8, BT), jnp.float32),
        scratch_types=scratch,
    )
    def k(table_hbm, idx_hbm, out_hbm, idxv, g0, g1, o0, o1, gs0, gs1, os0, os1):
        wid = lax.axis_index("s") * NUM_CORES + lax.axis_index("c")
        gb, ob, gs, osem = (g0, g1), (o0, o1), (gs0, gs1), (os0, os1)
        pltpu.sync_copy(idx_hbm.at[:, wid], idxv)
        iota = lax.iota(jnp.int32, 16)
        cvecs = [iota + 16 * q for q in range(D_MODEL // 16)]

        for p in range(2):
            pltpu.async_copy(table_hbm.at[idxv.at[0, p]], gb[p], gs[p])

        def outer(t2, _):
            for p in range(2):
                u = t2 * 2 + p
                rt, rs = u >> 3, u & 7
                pltpu.make_async_copy(
                    table_hbm.at[idxv.at[rt, rs]], gb[p], gs[p]
                ).wait()

                @pl.when(u >= 2)
                def _():
                    for i in range(8):
                        pltpu.make_async_copy(
                            ob[p].at[pl.ds(8 * i, 8), pl.ds(0, BT)],
                            out_hbm.at[0, i, 0],
                            osem[p],
                        ).wait()

                def body(r, _, p=p):
                    rv = jnp.full((16,), r, jnp.int32)
                    for q in range(D_MODEL // 16):
                        v = gb[p][r, pl.ds(16 * q, 16)]
                        plsc.store_scatter(ob[p], [cvecs[q], rv], v * SCALE)
                    return 0

                lax.fori_loop(0, BT, body, 0, unroll=2)
                for i in range(8):
                    pltpu.async_copy(
                        ob[p].at[pl.ds(8 * i, 8), pl.ds(0, BT)],
                        out_hbm.at[u, i, wid],
                        osem[p],
                    )

                uu = u + 2

                @pl.when(uu < n_units)
                def _():
                    pltpu.async_copy(
                        table_hbm.at[idxv.at[uu >> 3, uu & 7]], gb[p], gs[p]
                    )
            return 0

        lax.fori_loop(0, n_units // 2, outer, 0)
        for p in range(2):
            for i in range(8):
                pltpu.make_async_copy(
                    ob[p].at[pl.ds(8 * i, 8), pl.ds(0, BT)],
                    out_hbm.at[0, i, 0],
                    osem[p],
                ).wait()

    return k


def kernel(x, table):
    NB, NR = x.shape
    V = table.shape[0]
    JT, RT = NB // BT, NR // 8
    xt = x.T.reshape(RT, 8, JT, BT).transpose(0, 2, 1, 3).astype(jnp.int32)
    L = _make_embed(V, NB, NR)(table, xt)
    return L.transpose(2, 4, 0, 1, 3).reshape(NB, NR, D_MODEL)


# speedup vs baseline: 1.8759x; 1.0667x over previous
import functools
import math

import jax
import jax.numpy as jnp
from jax import lax
from jax.experimental import pallas as pl
from jax.experimental.pallas import tpu as pltpu
from jax.experimental.pallas import tpu_sc as plsc

D_MODEL = 64
SCALE = math.sqrt(D_MODEL)
NUM_CORES = 2
NUM_SUBCORES = 16
NW = NUM_CORES * NUM_SUBCORES
BT = 128
VBLK = 2048


def _repack(tableT):
    V = tableT.shape[1]
    grid = (V + VBLK - 1) // VBLK

    def body(in_ref, out_ref):
        t = in_ref[...].T.reshape(VBLK // 2, 2, D_MODEL)
        out_ref[...] = jnp.concatenate([t[:, 0, :], t[:, 1, :]], axis=1)

    return pl.pallas_call(
        body,
        grid=(grid,),
        in_specs=[pl.BlockSpec((D_MODEL, VBLK), lambda j: (0, j))],
        out_specs=pl.BlockSpec((VBLK // 2, 2 * D_MODEL), lambda j: (j, 0)),
        out_shape=jax.ShapeDtypeStruct((V // 2, 2 * D_MODEL), jnp.float32),
    )(tableT)


def _make_embed(V, NB, NR):
    JT = NB // BT
    RT = NR // 8
    n_units = NR
    mesh = plsc.VectorSubcoreMesh(core_axis_name="c", subcore_axis_name="s")

    OPAD = BT + 1
    scratch = (
        [pltpu.VMEM((RT, 8, BT), jnp.int32)]
        + [pltpu.VMEM((BT, D_MODEL), jnp.float32) for _ in range(2)]
        + [pltpu.VMEM((D_MODEL, OPAD), jnp.float32) for _ in range(2)]
        + [pltpu.SemaphoreType.DMA for _ in range(4)]
    )

    @functools.partial(
        pl.kernel,
        mesh=mesh,
        compiler_params=pltpu.CompilerParams(
            use_tc_tiling_on_sc=False, needs_layout_passes=False
        ),
        out_type=jax.ShapeDtypeStruct((NR, 8, JT, 8, BT), jnp.float32),
        scratch_types=scratch,
    )
    def k(table_hbm, idx_hbm, out_hbm, idxv, g0, g1, o0, o1, gs0, gs1, os0, os1):
        wid = lax.axis_index("s") * NUM_CORES + lax.axis_index("c")
        gb, ob, gs, osem = (g0, g1), (o0, o1), (gs0, gs1), (os0, os1)
        pltpu.sync_copy(idx_hbm.at[:, wid], idxv)
        iota = lax.iota(jnp.int32, 16)
        cvecs = [iota + 16 * q for q in range(D_MODEL // 16)]

        for p in range(2):
            pltpu.async_copy(table_hbm.at[idxv.at[0, p]], gb[p], gs[p])

        def outer(t2, _):
            for p in range(2):
                u = t2 * 2 + p
                rt, rs = u >> 3, u & 7
                pltpu.make_async_copy(
                    table_hbm.at[idxv.at[rt, rs]], gb[p], gs[p]
                ).wait()

                @pl.when(u >= 2)
                def _():
                    for i in range(8):
                        pltpu.make_async_copy(
                            ob[p].at[pl.ds(8 * i, 8), pl.ds(0, BT)],
                            out_hbm.at[0, i, 0],
                            osem[p],
                        ).wait()

                def body(r, _, p=p):
                    rv = jnp.full((16,), r, jnp.int32)
                    for q in range(D_MODEL // 16):
                        v = gb[p][r, pl.ds(16 * q, 16)]
                        plsc.store_scatter(ob[p], [cvecs[q], rv], v * SCALE)
                    return 0

                lax.fori_loop(0, BT, body, 0, unroll=4)
                for i in range(8):
                    pltpu.async_copy(
                        ob[p].at[pl.ds(8 * i, 8), pl.ds(0, BT)],
                        out_hbm.at[u, i, wid],
                        osem[p],
                    )

                uu = u + 2

                @pl.when(uu < n_units)
                def _():
                    pltpu.async_copy(
                        table_hbm.at[idxv.at[uu >> 3, uu & 7]], gb[p], gs[p]
                    )
            return 0

        lax.fori_loop(0, n_units // 2, outer, 0)
        for p in range(2):
            for i in range(8):
                pltpu.make_async_copy(
                    ob[p].at[pl.ds(8 * i, 8), pl.ds(0, BT)],
                    out_hbm.at[0, i, 0],
                    osem[p],
                ).wait()

    return k


def kernel(x, table):
    NB, NR = x.shape
    V = table.shape[0]
    JT, RT = NB // BT, NR // 8
    xt = x.T.reshape(RT, 8, JT, BT).transpose(0, 2, 1, 3).astype(jnp.int32)
    tab_lin = _repack(table.T).reshape(V, D_MODEL)
    L = _make_embed(V, NB, NR)(tab_lin, xt)
    return L.transpose(2, 4, 0, 1, 3).reshape(NB, NR, D_MODEL)


# speedup vs baseline: 2.7019x; 1.4403x over previous
import functools
import math

import jax
import jax.numpy as jnp
from jax import lax
from jax.experimental import pallas as pl
from jax.experimental.pallas import tpu as pltpu
from jax.experimental.pallas import tpu_sc as plsc

D_MODEL = 64
SCALE = math.sqrt(D_MODEL)
NUM_CORES = 2
NUM_SUBCORES = 16
NW = NUM_CORES * NUM_SUBCORES
BT = 128
VBLK = 2048


def _repack(tableT):
    V = tableT.shape[1]
    grid = (V + VBLK - 1) // VBLK

    def body(in_ref, out_ref):
        t = in_ref[...].T.reshape(VBLK // 2, 2, D_MODEL)
        out_ref[...] = jnp.concatenate([t[:, 0, :], t[:, 1, :]], axis=1)

    return pl.pallas_call(
        body,
        grid=(grid,),
        in_specs=[pl.BlockSpec((D_MODEL, VBLK), lambda j: (0, j))],
        out_specs=pl.BlockSpec((VBLK // 2, 2 * D_MODEL), lambda j: (j, 0)),
        out_shape=jax.ShapeDtypeStruct((V // 2, 2 * D_MODEL), jnp.float32),
    )(tableT)


def _make_embed(V, NB, NR):
    JT = NB // BT
    RT = NR // 8
    n_units = NR
    mesh = plsc.VectorSubcoreMesh(core_axis_name="c", subcore_axis_name="s")

    OPAD = BT + 1
    scratch = (
        [pltpu.VMEM((RT, 8, BT), jnp.int32)]
        + [pltpu.VMEM((BT, D_MODEL), jnp.float32) for _ in range(2)]
        + [pltpu.VMEM((D_MODEL, OPAD), jnp.float32) for _ in range(2)]
        + [pltpu.SemaphoreType.DMA for _ in range(4)]
    )

    @functools.partial(
        pl.kernel,
        mesh=mesh,
        compiler_params=pltpu.CompilerParams(
            use_tc_tiling_on_sc=False, needs_layout_passes=False
        ),
        out_type=jax.ShapeDtypeStruct((NR, 8, JT, 8, BT), jnp.float32),
        scratch_types=scratch,
    )
    def k(table_hbm, idx_hbm, out_hbm, idxv, g0, g1, o0, o1, gs0, gs1, os0, os1):
        wid = lax.axis_index("s") * NUM_CORES + lax.axis_index("c")
        gb, ob, gs, osem = (g0, g1), (o0, o1), (gs0, gs1), (os0, os1)
        pltpu.sync_copy(idx_hbm.at[:, wid], idxv)
        iota = lax.iota(jnp.int32, 16)
        cvecs = [iota + 16 * q for q in range(D_MODEL // 16)]

        for p in range(2):
            pltpu.async_copy(table_hbm.at[idxv.at[0, p]], gb[p], gs[p])

        def outer(t2, _):
            for p in range(2):
                u = t2 * 2 + p
                rt, rs = u >> 3, u & 7
                pltpu.make_async_copy(
                    table_hbm.at[idxv.at[rt, rs]], gb[p], gs[p]
                ).wait()

                @pl.when(u >= 2)
                def _():
                    for i in range(8):
                        pltpu.make_async_copy(
                            ob[p].at[pl.ds(8 * i, 8), pl.ds(0, BT)],
                            out_hbm.at[0, i, 0],
                            osem[p],
                        ).wait()

                @plsc.parallel_loop(0, BT, unroll=4)
                def _(r, p=p):
                    rv = jnp.full((16,), r, jnp.int32)
                    for q in range(D_MODEL // 16):
                        v = gb[p][r, pl.ds(16 * q, 16)]
                        plsc.store_scatter(ob[p], [cvecs[q], rv], v * SCALE)
                for i in range(8):
                    pltpu.async_copy(
                        ob[p].at[pl.ds(8 * i, 8), pl.ds(0, BT)],
                        out_hbm.at[u, i, wid],
                        osem[p],
                    )

                uu = u + 2

                @pl.when(uu < n_units)
                def _():
                    pltpu.async_copy(
                        table_hbm.at[idxv.at[uu >> 3, uu & 7]], gb[p], gs[p]
                    )
            return 0

        lax.fori_loop(0, n_units // 2, outer, 0)
        for p in range(2):
            for i in range(8):
                pltpu.make_async_copy(
                    ob[p].at[pl.ds(8 * i, 8), pl.ds(0, BT)],
                    out_hbm.at[0, i, 0],
                    osem[p],
                ).wait()

    return k


def kernel(x, table):
    NB, NR = x.shape
    V = table.shape[0]
    JT, RT = NB // BT, NR // 8
    xt = x.T.reshape(RT, 8, JT, BT).transpose(0, 2, 1, 3).astype(jnp.int32)
    tab_lin = _repack(table.T).reshape(V, D_MODEL)
    L = _make_embed(V, NB, NR)(tab_lin, xt)
    return L.transpose(2, 4, 0, 1, 3).reshape(NB, NR, D_MODEL)


# speedup vs baseline: 3.3260x; 1.2310x over previous
import functools
import math

import jax
import jax.numpy as jnp
from jax import lax
from jax.experimental import pallas as pl
from jax.experimental.pallas import tpu as pltpu
from jax.experimental.pallas import tpu_sc as plsc

D_MODEL = 64
SCALE = math.sqrt(D_MODEL)
NUM_CORES = 2
NUM_SUBCORES = 16
NW = NUM_CORES * NUM_SUBCORES
BT = 128
VBLK = 8192


def _repack(tableT):
    V = tableT.shape[1]
    grid = (V + VBLK - 1) // VBLK

    def body(in_ref, out_ref):
        t = in_ref[...].T.reshape(VBLK // 2, 2, D_MODEL)
        out_ref[...] = jnp.concatenate([t[:, 0, :], t[:, 1, :]], axis=1)

    return pl.pallas_call(
        body,
        grid=(grid,),
        in_specs=[pl.BlockSpec((D_MODEL, VBLK), lambda j: (0, j))],
        out_specs=pl.BlockSpec((VBLK // 2, 2 * D_MODEL), lambda j: (j, 0)),
        out_shape=jax.ShapeDtypeStruct((V // 2, 2 * D_MODEL), jnp.float32),
    )(tableT)


def _make_embed(V, NB, NR):
    JT = NB // BT
    RT = NR // 8
    n_units = NR
    mesh = plsc.VectorSubcoreMesh(core_axis_name="c", subcore_axis_name="s")

    OPAD = BT + 1
    scratch = (
        [pltpu.VMEM((RT, 8, BT), jnp.int32)]
        + [pltpu.VMEM((BT, D_MODEL), jnp.float32) for _ in range(2)]
        + [pltpu.VMEM((D_MODEL, OPAD), jnp.float32) for _ in range(2)]
        + [pltpu.SemaphoreType.DMA for _ in range(4)]
    )

    @functools.partial(
        pl.kernel,
        mesh=mesh,
        compiler_params=pltpu.CompilerParams(
            use_tc_tiling_on_sc=False, needs_layout_passes=False
        ),
        out_type=jax.ShapeDtypeStruct((NR, 8, JT, 8, BT), jnp.float32),
        scratch_types=scratch,
    )
    def k(table_hbm, idx_hbm, out_hbm, idxv, g0, g1, o0, o1, gs0, gs1, os0, os1):
        wid = lax.axis_index("s") * NUM_CORES + lax.axis_index("c")
        gb, ob, gs, osem = (g0, g1), (o0, o1), (gs0, gs1), (os0, os1)
        pltpu.sync_copy(idx_hbm.at[:, wid], idxv)
        iota = lax.iota(jnp.int32, 16)
        cvecs = [iota + 16 * q for q in range(D_MODEL // 16)]

        for p in range(2):
            pltpu.async_copy(table_hbm.at[idxv.at[0, p]], gb[p], gs[p])

        def outer(t2, _):
            for p in range(2):
                u = t2 * 2 + p
                rt, rs = u >> 3, u & 7
                pltpu.make_async_copy(
                    table_hbm.at[idxv.at[rt, rs]], gb[p], gs[p]
                ).wait()

                @pl.when(u >= 2)
                def _():
                    for i in range(8):
                        pltpu.make_async_copy(
                            ob[p].at[pl.ds(8 * i, 8), pl.ds(0, BT)],
                            out_hbm.at[0, i, 0],
                            osem[p],
                        ).wait()

                @plsc.parallel_loop(0, BT, unroll=4)
                def _(r, p=p):
                    rv = jnp.full((16,), r, jnp.int32)
                    for q in range(D_MODEL // 16):
                        v = gb[p][r, pl.ds(16 * q, 16)]
                        plsc.store_scatter(ob[p], [cvecs[q], rv], v * SCALE)
                for i in range(8):
                    pltpu.async_copy(
                        ob[p].at[pl.ds(8 * i, 8), pl.ds(0, BT)],
                        out_hbm.at[u, i, wid],
                        osem[p],
                    )

                uu = u + 2

                @pl.when(uu < n_units)
                def _():
                    pltpu.async_copy(
                        table_hbm.at[idxv.at[uu >> 3, uu & 7]], gb[p], gs[p]
                    )
            return 0

        lax.fori_loop(0, n_units // 2, outer, 0)
        for p in range(2):
            for i in range(8):
                pltpu.make_async_copy(
                    ob[p].at[pl.ds(8 * i, 8), pl.ds(0, BT)],
                    out_hbm.at[0, i, 0],
                    osem[p],
                ).wait()

    return k


def kernel(x, table):
    NB, NR = x.shape
    V = table.shape[0]
    JT, RT = NB // BT, NR // 8
    xt = x.T.reshape(RT, 8, JT, BT).transpose(0, 2, 1, 3).astype(jnp.int32)
    tab_lin = _repack(table.T).reshape(V, D_MODEL)
    L = _make_embed(V, NB, NR)(tab_lin, xt)
    return L.transpose(2, 4, 0, 1, 3).reshape(NB, NR, D_MODEL)


# speedup vs baseline: 3.3409x; 1.0045x over previous
import functools
import math

import jax
import jax.numpy as jnp
from jax import lax
from jax.experimental import pallas as pl
from jax.experimental.pallas import tpu as pltpu
from jax.experimental.pallas import tpu_sc as plsc

D_MODEL = 64
SCALE = math.sqrt(D_MODEL)
NUM_CORES = 2
NUM_SUBCORES = 16
NW = NUM_CORES * NUM_SUBCORES
BT = 128
VBLK = 16384


def _repack(tableT):
    V = tableT.shape[1]
    grid = (V + VBLK - 1) // VBLK

    def body(in_ref, out_ref):
        t = in_ref[...].T.reshape(VBLK // 2, 2, D_MODEL)
        out_ref[...] = jnp.concatenate([t[:, 0, :], t[:, 1, :]], axis=1)

    return pl.pallas_call(
        body,
        grid=(grid,),
        in_specs=[pl.BlockSpec((D_MODEL, VBLK), lambda j: (0, j))],
        out_specs=pl.BlockSpec((VBLK // 2, 2 * D_MODEL), lambda j: (j, 0)),
        out_shape=jax.ShapeDtypeStruct((V // 2, 2 * D_MODEL), jnp.float32),
    )(tableT)


def _make_embed(V, NB, NR):
    JT = NB // BT
    RT = NR // 8
    n_units = NR
    mesh = plsc.VectorSubcoreMesh(core_axis_name="c", subcore_axis_name="s")

    OPAD = BT + 1
    scratch = (
        [pltpu.VMEM((RT, 8, BT), jnp.int32)]
        + [pltpu.VMEM((BT, D_MODEL), jnp.float32) for _ in range(2)]
        + [pltpu.VMEM((D_MODEL, OPAD), jnp.float32) for _ in range(2)]
        + [pltpu.SemaphoreType.DMA for _ in range(4)]
    )

    @functools.partial(
        pl.kernel,
        mesh=mesh,
        compiler_params=pltpu.CompilerParams(
            use_tc_tiling_on_sc=False, needs_layout_passes=False
        ),
        out_type=jax.ShapeDtypeStruct((NR, 8, JT, 8, BT), jnp.float32),
        scratch_types=scratch,
    )
    def k(table_hbm, idx_hbm, out_hbm, idxv, g0, g1, o0, o1, gs0, gs1, os0, os1):
        wid = lax.axis_index("s") * NUM_CORES + lax.axis_index("c")
        gb, ob, gs, osem = (g0, g1), (o0, o1), (gs0, gs1), (os0, os1)
        pltpu.sync_copy(idx_hbm.at[:, wid], idxv)
        iota = lax.iota(jnp.int32, 16)
        cvecs = [iota + 16 * q for q in range(D_MODEL // 16)]

        for p in range(2):
            pltpu.async_copy(table_hbm.at[idxv.at[0, p]], gb[p], gs[p])

        def outer(t2, _):
            for p in range(2):
                u = t2 * 2 + p
                rt, rs = u >> 3, u & 7
                pltpu.make_async_copy(
                    table_hbm.at[idxv.at[rt, rs]], gb[p], gs[p]
                ).wait()

                @pl.when(u >= 2)
                def _():
                    for i in range(8):
                        pltpu.make_async_copy(
                            ob[p].at[pl.ds(8 * i, 8), pl.ds(0, BT)],
                            out_hbm.at[0, i, 0],
                            osem[p],
                        ).wait()

                @plsc.parallel_loop(0, BT, unroll=4)
                def _(r, p=p):
                    rv = jnp.full((16,), r, jnp.int32)
                    for q in range(D_MODEL // 16):
                        v = gb[p][r, pl.ds(16 * q, 16)]
                        plsc.store_scatter(ob[p], [cvecs[q], rv], v * SCALE)
                for i in range(8):
                    pltpu.async_copy(
                        ob[p].at[pl.ds(8 * i, 8), pl.ds(0, BT)],
                        out_hbm.at[u, i, wid],
                        osem[p],
                    )

                uu = u + 2

                @pl.when(uu < n_units)
                def _():
                    pltpu.async_copy(
                        table_hbm.at[idxv.at[uu >> 3, uu & 7]], gb[p], gs[p]
                    )
            return 0

        lax.fori_loop(0, n_units // 2, outer, 0)
        for p in range(2):
            for i in range(8):
                pltpu.make_async_copy(
                    ob[p].at[pl.ds(8 * i, 8), pl.ds(0, BT)],
                    out_hbm.at[0, i, 0],
                    osem[p],
                ).wait()

    return k


def kernel(x, table):
    NB, NR = x.shape
    V = table.shape[0]
    JT, RT = NB // BT, NR // 8
    xt = x.T.reshape(RT, 8, JT, BT).transpose(0, 2, 1, 3).astype(jnp.int32)
    tab_lin = _repack(table.T).reshape(V, D_MODEL)
    L = _make_embed(V, NB, NR)(tab_lin, xt)
    return L.transpose(2, 4, 0, 1, 3).reshape(NB, NR, D_MODEL)
